# baseline (speedup 1.0000x reference)
import functools

import jax
import jax.numpy as jnp
from jax import lax
from jax.experimental import pallas as pl
from jax.experimental.pallas import tpu as pltpu
from jax.experimental.pallas import tpu_sc as plsc

HIDDEN_SIZE = 512
NUM_SCENARIOS = 8
BATCH = 4096
NUM_CORES = 2
NUM_SUBCORES = 16
NUM_WORKERS = NUM_CORES * NUM_SUBCORES
B_PER_W = BATCH // NUM_WORKERS
LANES = 16
VPR = HIDDEN_SIZE // LANES
NCHUNK = 8
CHUNK = B_PER_W // NCHUNK

_mesh = plsc.VectorSubcoreMesh(core_axis_name="c", subcore_axis_name="s")


@functools.partial(
    pl.kernel,
    mesh=_mesh,
    out_type=jax.ShapeDtypeStruct((BATCH, HIDDEN_SIZE), jnp.float32),
    scratch_types=[
        pltpu.VMEM((B_PER_W,), jnp.int32),
        pltpu.VMEM((NUM_SCENARIOS, HIDDEN_SIZE), jnp.float32),
        pltpu.VMEM((B_PER_W, HIDDEN_SIZE), jnp.float32),
        pltpu.SMEM((B_PER_W,), jnp.int32),
        pltpu.SemaphoreType.DMA,
        pltpu.SemaphoreType.DMA((NCHUNK,)),
    ],
)
def _gather_kernel(idx_hbm, table_hbm, out_hbm, idx_v, tbl_v, rows_v, idx_s, tsem, wsem):
    wid = lax.axis_index("s") * NUM_CORES + lax.axis_index("c")
    base = wid * B_PER_W
    tbl_copy = pltpu.async_copy(table_hbm, tbl_v, tsem)
    pltpu.sync_copy(idx_hbm.at[pl.ds(base, B_PER_W)], idx_v)

    for g in range(B_PER_W // LANES):
        vec = idx_v[pl.ds(g * LANES, LANES)]
        for l in range(LANES):
            idx_s[g * LANES + l] = vec[l]

    tbl_copy.wait()

    writes = []
    for ch in range(NCHUNK):

        @plsc.parallel_loop(ch * CHUNK, (ch + 1) * CHUNK, unroll=4)
        def _(j):
            r = idx_s[j]
            for c in range(VPR):
                rows_v[j, pl.ds(c * LANES, LANES)] = tbl_v[r, pl.ds(c * LANES, LANES)]

        writes.append(
            pltpu.async_copy(
                rows_v.at[pl.ds(ch * CHUNK, CHUNK)],
                out_hbm.at[pl.ds(base + ch * CHUNK, CHUNK)],
                wsem.at[ch],
            )
        )
    for w in writes:
        w.wait()


def kernel(scenarios, table):
    return _gather_kernel(scenarios.astype(jnp.int32), table)


# speedup vs baseline: 1.2908x; 1.2908x over previous
import functools

import jax
import jax.numpy as jnp
from jax import lax
from jax.experimental import pallas as pl
from jax.experimental.pallas import tpu as pltpu
from jax.experimental.pallas import tpu_sc as plsc

HIDDEN_SIZE = 512
NUM_SCENARIOS = 8
BATCH = 4096
NUM_CORES = 2
NUM_SUBCORES = 16
NUM_WORKERS = NUM_CORES * NUM_SUBCORES
B_PER_W = BATCH // NUM_WORKERS
LANES = 16
VPR = HIDDEN_SIZE // LANES
NCHUNK = 4
CHUNK = B_PER_W // NCHUNK

_mesh = plsc.VectorSubcoreMesh(core_axis_name="c", subcore_axis_name="s")


@functools.partial(
    pl.kernel,
    mesh=_mesh,
    out_type=jax.ShapeDtypeStruct((BATCH, HIDDEN_SIZE), jnp.float32),
    scratch_types=[
        pltpu.VMEM((B_PER_W,), jnp.int32),
        pltpu.VMEM((NUM_SCENARIOS, HIDDEN_SIZE), jnp.float32),
        pltpu.VMEM((B_PER_W, HIDDEN_SIZE), jnp.float32),
        pltpu.SMEM((B_PER_W,), jnp.int32),
        pltpu.SemaphoreType.DMA,
        pltpu.SemaphoreType.DMA((NCHUNK,)),
    ],
)
def _gather_kernel(idx_hbm, table_hbm, out_hbm, idx_v, tbl_v, rows_v, idx_s, tsem, wsem):
    wid = lax.axis_index("s") * NUM_CORES + lax.axis_index("c")
    base = wid * B_PER_W
    tbl_copy = pltpu.async_copy(table_hbm, tbl_v, tsem)
    pltpu.sync_copy(idx_hbm.at[pl.ds(base, B_PER_W)], idx_v)

    for g in range(B_PER_W // LANES):
        vec = idx_v[pl.ds(g * LANES, LANES)]
        for l in range(LANES):
            idx_s[g * LANES + l] = vec[l]

    tbl_copy.wait()

    writes = []
    for ch in range(NCHUNK):

        @plsc.parallel_loop(ch * CHUNK, (ch + 1) * CHUNK, unroll=2)
        def _(j):
            r = idx_s[j]
            for c in range(VPR):
                rows_v[j, pl.ds(c * LANES, LANES)] = tbl_v[r, pl.ds(c * LANES, LANES)]

        writes.append(
            pltpu.async_copy(
                rows_v.at[pl.ds(ch * CHUNK, CHUNK)],
                out_hbm.at[pl.ds(base + ch * CHUNK, CHUNK)],
                wsem.at[ch],
            )
        )
    for w in writes:
        w.wait()


def kernel(scenarios, table):
    return _gather_kernel(scenarios.astype(jnp.int32), table)


# speedup vs baseline: 1.3588x; 1.0527x over previous
import functools

import jax
import jax.numpy as jnp
from jax import lax
from jax.experimental import pallas as pl
from jax.experimental.pallas import tpu as pltpu
from jax.experimental.pallas import tpu_sc as plsc

HIDDEN_SIZE = 512
NUM_SCENARIOS = 8
BATCH = 4096
NUM_CORES = 2
NUM_SUBCORES = 16
NUM_WORKERS = NUM_CORES * NUM_SUBCORES
B_PER_W = BATCH // NUM_WORKERS
LANES = 16
VPR = HIDDEN_SIZE // LANES
NCHUNK = 2
CHUNK = B_PER_W // NCHUNK

_mesh = plsc.VectorSubcoreMesh(core_axis_name="c", subcore_axis_name="s")


@functools.partial(
    pl.kernel,
    mesh=_mesh,
    out_type=jax.ShapeDtypeStruct((BATCH, HIDDEN_SIZE), jnp.float32),
    scratch_types=[
        pltpu.VMEM((B_PER_W,), jnp.int32),
        pltpu.VMEM((NUM_SCENARIOS, HIDDEN_SIZE), jnp.float32),
        pltpu.VMEM((B_PER_W, HIDDEN_SIZE), jnp.float32),
        pltpu.SMEM((B_PER_W,), jnp.int32),
        pltpu.SemaphoreType.DMA,
        pltpu.SemaphoreType.DMA((NCHUNK,)),
    ],
)
def _gather_kernel(idx_hbm, table_hbm, out_hbm, idx_v, tbl_v, rows_v, idx_s, tsem, wsem):
    wid = lax.axis_index("s") * NUM_CORES + lax.axis_index("c")
    base = wid * B_PER_W
    tbl_copy = pltpu.async_copy(table_hbm, tbl_v, tsem)
    pltpu.sync_copy(idx_hbm.at[pl.ds(base, B_PER_W)], idx_v)

    for g in range(B_PER_W // LANES):
        vec = idx_v[pl.ds(g * LANES, LANES)]
        for l in range(LANES):
            idx_s[g * LANES + l] = vec[l]

    tbl_copy.wait()

    writes = []
    for ch in range(NCHUNK):

        @plsc.parallel_loop(ch * CHUNK, (ch + 1) * CHUNK, unroll=2)
        def _(j):
            r = idx_s[j]
            for c in range(VPR):
                rows_v[j, pl.ds(c * LANES, LANES)] = tbl_v[r, pl.ds(c * LANES, LANES)]

        writes.append(
            pltpu.async_copy(
                rows_v.at[pl.ds(ch * CHUNK, CHUNK)],
                out_hbm.at[pl.ds(base + ch * CHUNK, CHUNK)],
                wsem.at[ch],
            )
        )
    for w in writes:
        w.wait()


def kernel(scenarios, table):
    return _gather_kernel(scenarios.astype(jnp.int32), table)
